# baseline (speedup 1.0000x reference)
import functools
import jax
import jax.numpy as jnp
from jax import lax
from jax.experimental import pallas as pl
from jax.experimental.pallas import tpu as pltpu
from jax.experimental.pallas import tpu_sc as plsc

_NC = 2
_NS = 16
_NW = _NC * _NS

_B = 16384
_D = 64
_CHUNK = 128
_B_PER_W = _B // _NW
_NCH = _B_PER_W // _CHUNK


def _sc_gather_body(user_hbm, items_hbm, utab_hbm, itab_hbm,
                    uout_hbm, iout_hbm,
                    uidx_v, iidx_v, urows_v, irows_v, usem, isem):
  wid = lax.axis_index("s") * _NC + lax.axis_index("c")
  pltpu.sync_copy(user_hbm.at[wid], uidx_v)
  pltpu.sync_copy(items_hbm.at[wid], iidx_v)
  copies = []
  for j in range(_NCH):
    copies.append(pltpu.async_copy(utab_hbm.at[uidx_v.at[j]], urows_v.at[j], usem))
    copies.append(pltpu.async_copy(itab_hbm.at[iidx_v.at[j]], irows_v.at[j], isem))
  for c in copies:
    c.wait()
  pltpu.sync_copy(urows_v, uout_hbm.at[wid])
  pltpu.sync_copy(irows_v, iout_hbm.at[wid])


@jax.jit
def _sc_gather(user_r, items_r, user_table, item_table):
  mesh = plsc.VectorSubcoreMesh(core_axis_name="c", subcore_axis_name="s")
  out = jax.ShapeDtypeStruct((_NW, _NCH, _CHUNK, _D), jnp.float32)
  k = functools.partial(
      pl.kernel, mesh=mesh,
      compiler_params=pltpu.CompilerParams(use_tc_tiling_on_sc=False),
      out_type=(out, out),
      scratch_types=[
          pltpu.VMEM((_NCH, _CHUNK), jnp.int32),
          pltpu.VMEM((_NCH, _CHUNK), jnp.int32),
          pltpu.VMEM((_NCH, _CHUNK, _D), jnp.float32),
          pltpu.VMEM((_NCH, _CHUNK, _D), jnp.float32),
          pltpu.SemaphoreType.DMA,
          pltpu.SemaphoreType.DMA,
      ],
  )(_sc_gather_body)
  return k(user_r, items_r, user_table, item_table)


def _mlp_body(u_ref, i_ref, w1u_ref, w1i_ref, b1_ref, w2_ref, b2_ref,
              w3_ref, b3_ref, w4_ref, b4_ref, out_ref):
  h = jnp.dot(u_ref[...], w1u_ref[...], preferred_element_type=jnp.float32)
  h += jnp.dot(i_ref[...], w1i_ref[...], preferred_element_type=jnp.float32)
  h = jnp.maximum(h + b1_ref[...], 0.0)
  h = jnp.dot(h, w2_ref[...], preferred_element_type=jnp.float32)
  h = jnp.maximum(h + b2_ref[...], 0.0)
  h = jnp.dot(h, w3_ref[...], preferred_element_type=jnp.float32)
  h = jnp.maximum(h + b3_ref[...], 0.0)
  out_ref[...] = (jnp.dot(h, w4_ref[...], preferred_element_type=jnp.float32)
                  + b4_ref[...])


_BB = 2048


@jax.jit
def _tc_mlp(ue, ie, w1u, w1i, b1, w2, b2, w3, b3, w4, b4):
  nb = _B // _BB
  full = lambda shape: pl.BlockSpec(shape, lambda i: (0, 0))
  return pl.pallas_call(
      _mlp_body,
      grid=(nb,),
      in_specs=[
          pl.BlockSpec((_BB, _D), lambda i: (i, 0)),
          pl.BlockSpec((_BB, _D), lambda i: (i, 0)),
          full((_D, 32)), full((_D, 32)), full((1, 32)),
          full((32, 16)), full((1, 16)),
          full((16, 8)), full((1, 8)),
          full((8, 1)), full((1, 1)),
      ],
      out_specs=pl.BlockSpec((_BB, 1), lambda i: (i, 0)),
      out_shape=jax.ShapeDtypeStruct((_B, 1), jnp.float32),
  )(ue, ie, w1u, w1i, b1, w2, b2, w3, b3, w4, b4)


def kernel(user, items, user_table, item_table, W1, b1, W2, b2, W3, b3, W4, b4):
  user_r = user.astype(jnp.int32).reshape(_NW, _NCH, _CHUNK)
  items_r = items.astype(jnp.int32).reshape(_NW, _NCH, _CHUNK)
  ue, ie = _sc_gather(user_r, items_r, user_table, item_table)
  ue = ue.reshape(_B, _D)
  ie = ie.reshape(_B, _D)
  w1u = W1[:, :_D].T
  w1i = W1[:, _D:].T
  return _tc_mlp(ue, ie, w1u, w1i, b1.reshape(1, 32),
                 W2.T, b2.reshape(1, 16), W3.T, b3.reshape(1, 8),
                 W4.T, b4.reshape(1, 1))


# speedup vs baseline: 1.0016x; 1.0016x over previous
import functools
import jax
import jax.numpy as jnp
from jax import lax
from jax.experimental import pallas as pl
from jax.experimental.pallas import tpu as pltpu
from jax.experimental.pallas import tpu_sc as plsc

_NC = 2
_NS = 16
_NW = _NC * _NS

_B = 16384
_D = 64
_BPW = _B // _NW
_CHUNK = 128
_NCH = _BPW // _CHUNK
_IDXROWS = _B // _CHUNK

_BLK = 2048


def _sc_gather_body(uidx_hbm, iidx_hbm, utab_hbm, itab_hbm, out_hbm,
                    uidx_v, iidx_v, urows_v, irows_v, sem):
  wid = lax.axis_index("s") * _NC + lax.axis_index("c")
  base = wid * _BPW
  row0 = wid * _NCH
  pltpu.sync_copy(uidx_hbm.at[pl.ds(row0, _NCH)], uidx_v)
  pltpu.sync_copy(iidx_hbm.at[pl.ds(row0, _NCH)], iidx_v)

  copies = []
  for ch in range(_NCH):
    sl = pl.ds(ch * _CHUNK, _CHUNK)
    copies.append(
        pltpu.async_copy(utab_hbm.at[uidx_v.at[ch]], urows_v.at[sl], sem))
    copies.append(
        pltpu.async_copy(itab_hbm.at[iidx_v.at[ch]], irows_v.at[sl], sem))
  for c in copies:
    c.wait()

  pltpu.sync_copy(urows_v, out_hbm.at[0, pl.ds(base, _BPW)])
  pltpu.sync_copy(irows_v, out_hbm.at[1, pl.ds(base, _BPW)])


@jax.jit
def _sc_gather(uidx2, iidx2, utab, itab):
  mesh = plsc.VectorSubcoreMesh(core_axis_name="c", subcore_axis_name="s")
  k = functools.partial(
      pl.kernel, mesh=mesh,
      compiler_params=pltpu.CompilerParams(use_tc_tiling_on_sc=False),
      out_type=jax.ShapeDtypeStruct((2, _B, _D), jnp.float32),
      scratch_types=[
          pltpu.VMEM((_NCH, _CHUNK), jnp.int32),
          pltpu.VMEM((_NCH, _CHUNK), jnp.int32),
          pltpu.VMEM((_BPW, _D), jnp.float32),
          pltpu.VMEM((_BPW, _D), jnp.float32),
          pltpu.SemaphoreType.DMA,
      ],
  )(_sc_gather_body)
  return k(uidx2, iidx2, utab, itab)


def _mlp_body(x_ref, w1u_ref, w1v_ref, b1_ref, w2_ref, b2_ref,
              w3_ref, b3_ref, w4_ref, b4_ref, out_ref):
  u = x_ref[0]
  v = x_ref[1]
  h = jnp.dot(u, w1u_ref[...], preferred_element_type=jnp.float32)
  h = h + jnp.dot(v, w1v_ref[...], preferred_element_type=jnp.float32)
  h = jnp.maximum(h + b1_ref[...], 0.0)
  h = jnp.dot(h, w2_ref[...], preferred_element_type=jnp.float32)
  h = jnp.maximum(h + b2_ref[...], 0.0)
  h = jnp.dot(h, w3_ref[...], preferred_element_type=jnp.float32)
  h = jnp.maximum(h + b3_ref[...], 0.0)
  out_ref[...] = (jnp.dot(h, w4_ref[...], preferred_element_type=jnp.float32)
                  + b4_ref[...])


@jax.jit
def _tc_mlp(x, w1u, w1v, b1, w2, b2, w3, b3, w4, b4):
  full = lambda shape: pl.BlockSpec(shape, lambda i: tuple(0 for _ in shape))
  return pl.pallas_call(
      _mlp_body,
      grid=(_B // _BLK,),
      in_specs=[
          pl.BlockSpec((2, _BLK, _D), lambda i: (0, i, 0)),
          full((_D, 32)), full((_D, 32)), full((1, 32)),
          full((32, 16)), full((1, 16)),
          full((16, 8)), full((1, 8)),
          full((8, 1)), full((1, 1)),
      ],
      out_specs=pl.BlockSpec((_BLK, 1), lambda i: (i, 0)),
      out_shape=jax.ShapeDtypeStruct((_B, 1), jnp.float32),
  )(x, w1u, w1v, b1, w2, b2, w3, b3, w4, b4)


def kernel(user, items, user_table, item_table, W1, b1, W2, b2, W3, b3, W4, b4):
  u2 = user.astype(jnp.int32).reshape(_IDXROWS, _CHUNK)
  i2 = items.astype(jnp.int32).reshape(_IDXROWS, _CHUNK)
  x = _sc_gather(u2, i2, user_table, item_table)
  return _tc_mlp(x, W1[:, :_D].T, W1[:, _D:].T, b1.reshape(1, 32),
                 W2.T, b2.reshape(1, 16), W3.T, b3.reshape(1, 8),
                 W4.T, b4.reshape(1, 1))


# speedup vs baseline: 1.0041x; 1.0025x over previous
import functools
import jax
import jax.numpy as jnp
from jax import lax
from jax.experimental import pallas as pl
from jax.experimental.pallas import tpu as pltpu
from jax.experimental.pallas import tpu_sc as plsc

_NC = 2
_NS = 16
_NW = _NC * _NS

_B = 16384
_D = 64
_BPW = _B // _NW
_CHUNK = 128
_NCH = _BPW // _CHUNK
_IDXROWS = _B // _CHUNK

_BLK = 2048


def _sc_gather_body(uidx_hbm, iidx_hbm, utab_hbm, itab_hbm, out_hbm,
                    uidx_v, iidx_v, urows_v, irows_v, sem):
  wid = lax.axis_index("s") * _NC + lax.axis_index("c")
  base = wid * _BPW
  row0 = wid * _NCH
  pltpu.sync_copy(uidx_hbm.at[pl.ds(row0, _NCH)], uidx_v)
  pltpu.sync_copy(iidx_hbm.at[pl.ds(row0, _NCH)], iidx_v)

  copies = []
  for ch in range(_NCH):
    sl = pl.ds(ch * _CHUNK, _CHUNK)
    copies.append(
        pltpu.async_copy(utab_hbm.at[uidx_v.at[ch]], urows_v.at[sl], sem))
    copies.append(
        pltpu.async_copy(itab_hbm.at[iidx_v.at[ch]], irows_v.at[sl], sem))
  for c in copies:
    c.wait()

  pltpu.sync_copy(urows_v, out_hbm.at[0, pl.ds(base, _BPW)])
  pltpu.sync_copy(irows_v, out_hbm.at[1, pl.ds(base, _BPW)])


def _sc_gather(uidx2, iidx2, utab, itab):
  mesh = plsc.VectorSubcoreMesh(core_axis_name="c", subcore_axis_name="s")
  k = functools.partial(
      pl.kernel, mesh=mesh,
      compiler_params=pltpu.CompilerParams(use_tc_tiling_on_sc=False),
      out_type=jax.ShapeDtypeStruct((2, _B, _D), jnp.float32),
      scratch_types=[
          pltpu.VMEM((_NCH, _CHUNK), jnp.int32),
          pltpu.VMEM((_NCH, _CHUNK), jnp.int32),
          pltpu.VMEM((_BPW, _D), jnp.float32),
          pltpu.VMEM((_BPW, _D), jnp.float32),
          pltpu.SemaphoreType.DMA,
      ],
  )(_sc_gather_body)
  return k(uidx2, iidx2, utab, itab)


def _mlp_body(x_ref, w1u_ref, w1v_ref, b1_ref, w2_ref, b2_ref,
              w3_ref, b3_ref, w4_ref, b4_ref, out_ref):
  u = x_ref[0]
  v = x_ref[1]
  h = jnp.dot(u, w1u_ref[...], preferred_element_type=jnp.float32)
  h = h + jnp.dot(v, w1v_ref[...], preferred_element_type=jnp.float32)
  h = jnp.maximum(h + b1_ref[...], 0.0)
  h = jnp.dot(h, w2_ref[...], preferred_element_type=jnp.float32)
  h = jnp.maximum(h + b2_ref[...], 0.0)
  h = jnp.dot(h, w3_ref[...], preferred_element_type=jnp.float32)
  h = jnp.maximum(h + b3_ref[...], 0.0)
  out_ref[...] = (jnp.dot(h, w4_ref[...], preferred_element_type=jnp.float32)
                  + b4_ref[...])


def _tc_mlp(x, w1u, w1v, b1, w2, b2, w3, b3, w4, b4):
  full = lambda shape: pl.BlockSpec(shape, lambda i: tuple(0 for _ in shape))
  return pl.pallas_call(
      _mlp_body,
      grid=(_B // _BLK,),
      in_specs=[
          pl.BlockSpec((2, _BLK, _D), lambda i: (0, i, 0)),
          full((_D, 32)), full((_D, 32)), full((1, 32)),
          full((32, 16)), full((1, 16)),
          full((16, 8)), full((1, 8)),
          full((8, 1)), full((1, 1)),
      ],
      out_specs=pl.BlockSpec((_BLK, 1), lambda i: (i, 0)),
      out_shape=jax.ShapeDtypeStruct((_B, 1), jnp.float32),
  )(x, w1u, w1v, b1, w2, b2, w3, b3, w4, b4)


@jax.jit
def kernel(user, items, user_table, item_table, W1, b1, W2, b2, W3, b3, W4, b4):
  u2 = user.astype(jnp.int32).reshape(_IDXROWS, _CHUNK)
  i2 = items.astype(jnp.int32).reshape(_IDXROWS, _CHUNK)
  x = _sc_gather(u2, i2, user_table, item_table)
  return _tc_mlp(x, W1[:, :_D].T, W1[:, _D:].T, b1.reshape(1, 32),
                 W2.T, b2.reshape(1, 16), W3.T, b3.reshape(1, 8),
                 W4.T, b4.reshape(1, 1))
